# baseline (speedup 1.0000x reference)
import functools

import jax
import jax.numpy as jnp
from jax import lax
from jax.experimental import pallas as pl
from jax.experimental.pallas import tpu as pltpu
from jax.experimental.pallas import tpu_sc as plsc

CORES = 2
TILES = 16
LANES = 16
CH = 128

N = 10000
D = 256
E = 160000
BLK = 1000

_KJC = -(-E // (TILES * CH))
KJ = _KJC + (_KJC % 2)
EPT = KJ * CH
EP = EPT * TILES
NROWS = 10240
SLAB = NROWS // TILES
ZR = 32
HALF = D // 2


def _zero_vmem(buf, nrows, value=0.0):
    def fill(i, carry):
        for k in range(HALF // LANES):
            buf[i, pl.ds(k * LANES, LANES)] = jnp.full(
                (LANES,), value, jnp.float32)
        return carry
    lax.fori_loop(0, nrows, fill, 0)


def _deg_body(e_ref, out_ref, idx_v, ones_v, zbuf, deg_sh, sem):
    c = lax.axis_index("c")
    s = lax.axis_index("s")
    _zero_vmem(zbuf, ZR)
    _zero_vmem(ones_v, CH, 1.0)

    for t in range(SLAB // ZR):
        pltpu.sync_copy(zbuf, deg_sh.at[pl.ds(s * SLAB + t * ZR, ZR)])
    plsc.subcore_barrier()

    pltpu.sync_copy(e_ref.at[c, s], idx_v)

    def batch(b, carry):
        for k in range(8):
            pltpu.async_copy(ones_v, deg_sh.at[idx_v.at[b * 8 + k]], sem,
                             add=True)
        for k in range(8):
            pltpu.make_async_copy(ones_v, deg_sh.at[idx_v.at[b * 8 + k]],
                                  sem).wait()
        return carry
    lax.fori_loop(0, KJ // 8, batch, 0)

    plsc.subcore_barrier()
    pltpu.sync_copy(deg_sh.at[pl.ds(s * SLAB, SLAB)],
                    out_ref.at[c, pl.ds(s * SLAB, SLAB)])


@functools.cache
def _deg_kernel():
    mesh = plsc.VectorSubcoreMesh(
        core_axis_name="c", subcore_axis_name="s",
        num_cores=CORES, num_subcores=TILES)
    return pl.kernel(
        _deg_body,
        out_type=jax.ShapeDtypeStruct((CORES, NROWS, HALF), jnp.float32),
        mesh=mesh,
        scratch_types=[
            pltpu.VMEM((KJ, CH), jnp.int32),
            pltpu.VMEM((CH, HALF), jnp.float32),
            pltpu.VMEM((ZR, HALF), jnp.float32),
            pltpu.VMEM_SHARED((NROWS, HALF), jnp.float32),
            pltpu.SemaphoreType.DMA,
        ],
    )


def _scale_body(xa_ref, xb_ref, dego_ref, y0_ref, y1_ref):
    s = lax.rsqrt(jnp.maximum(dego_ref[...], 1.0))
    y0_ref[...] = xa_ref[...] * s
    y1_ref[...] = xb_ref[...] * s


_scale_kernel = pl.pallas_call(
    _scale_body,
    grid=(N // BLK,),
    in_specs=[
        pl.BlockSpec((BLK, HALF), lambda i: (i, 0)),
        pl.BlockSpec((BLK, HALF), lambda i: (i, 1)),
        pl.BlockSpec((BLK, 1), lambda i: (i, 0)),
    ],
    out_specs=[
        pl.BlockSpec((BLK, HALF), lambda i: (i, 0)),
        pl.BlockSpec((BLK, HALF), lambda i: (i, 0)),
    ],
    out_shape=[
        jax.ShapeDtypeStruct((N, HALF), jnp.float32),
        jax.ShapeDtypeStruct((N, HALF), jnp.float32),
    ],
)


def _agg_body(y_ref, src_ref, dst_ref, out_ref,
              src_v, dst_v, rows_v, zbuf, agg_sh, sem):
    c = lax.axis_index("c")
    s = lax.axis_index("s")

    _zero_vmem(zbuf, ZR)
    for t in range(SLAB // ZR):
        pltpu.sync_copy(zbuf, agg_sh.at[pl.ds(s * SLAB + t * ZR, ZR)])
    plsc.subcore_barrier()

    pltpu.sync_copy(src_ref.at[c, s], src_v)
    pltpu.sync_copy(dst_ref.at[s], dst_v)

    def chunk(j, carry):
        pltpu.async_copy(y_ref.at[src_v.at[j]], rows_v, sem).wait()
        pltpu.sync_copy(rows_v, agg_sh.at[dst_v.at[j]], add=True)
        return carry
    lax.fori_loop(0, KJ, chunk, 0)

    plsc.subcore_barrier()
    pltpu.sync_copy(agg_sh.at[pl.ds(s * SLAB, SLAB)],
                    out_ref.at[c, pl.ds(s * SLAB, SLAB)])


@functools.cache
def _agg_kernel():
    mesh = plsc.VectorSubcoreMesh(
        core_axis_name="c", subcore_axis_name="s",
        num_cores=CORES, num_subcores=TILES)
    return pl.kernel(
        _agg_body,
        out_type=jax.ShapeDtypeStruct((CORES, NROWS, HALF), jnp.float32),
        mesh=mesh,
        scratch_types=[
            pltpu.VMEM((KJ, CH), jnp.int32),
            pltpu.VMEM((KJ, CH), jnp.int32),
            pltpu.VMEM((CH, HALF), jnp.float32),
            pltpu.VMEM((ZR, HALF), jnp.float32),
            pltpu.VMEM_SHARED((NROWS, HALF), jnp.float32),
            pltpu.SemaphoreType.DMA,
        ],
    )


def _xw2_body(x_ref, w2_ref, b1_ref, out_ref):
    out_ref[...] = jnp.dot(x_ref[...], w2_ref[...],
                           preferred_element_type=jnp.float32) + b1_ref[...]


def _make_xw2(H):
    return pl.pallas_call(
        _xw2_body,
        grid=(N // BLK,),
        in_specs=[
            pl.BlockSpec((BLK, D), lambda i: (i, 0)),
            pl.BlockSpec((D, H), lambda i: (0, 0)),
            pl.BlockSpec((1, H), lambda i: (0, 0)),
        ],
        out_specs=pl.BlockSpec((BLK, H), lambda i: (i, 0)),
        out_shape=jax.ShapeDtypeStruct((N, H), jnp.float32),
    )


def _dense_body(xw_ref, a0_ref, a1_ref, degi_ref, w1a_ref, w1b_ref,
                w3_ref, b3_ref, out_ref):
    s = lax.rsqrt(jnp.maximum(degi_ref[...], 1.0))
    h = jnp.dot(a0_ref[0] * s, w1a_ref[...],
                preferred_element_type=jnp.float32)
    h += jnp.dot(a1_ref[0] * s, w1b_ref[...],
                 preferred_element_type=jnp.float32)
    h += xw_ref[...]
    h = jnp.where(h >= 0.0, h, 0.2 * h)
    logits = jnp.dot(h, w3_ref[...], preferred_element_type=jnp.float32)
    logits += b3_ref[...]
    m = jnp.max(logits, axis=-1, keepdims=True)
    e = jnp.exp(logits - m)
    out_ref[...] = e / jnp.sum(e, axis=-1, keepdims=True)


def _make_dense(H, LBL):
    return pl.pallas_call(
        _dense_body,
        grid=(N // BLK,),
        in_specs=[
            pl.BlockSpec((BLK, H), lambda i: (i, 0)),
            pl.BlockSpec((1, BLK, HALF), lambda i: (0, i, 0)),
            pl.BlockSpec((1, BLK, HALF), lambda i: (1, i, 0)),
            pl.BlockSpec((BLK, 1), lambda i: (i, 0)),
            pl.BlockSpec((HALF, H), lambda i: (0, 0)),
            pl.BlockSpec((HALF, H), lambda i: (1, 0)),
            pl.BlockSpec((H, LBL), lambda i: (0, 0)),
            pl.BlockSpec((1, LBL), lambda i: (0, 0)),
        ],
        out_specs=pl.BlockSpec((BLK, LBL), lambda i: (i, 0)),
        out_shape=jax.ShapeDtypeStruct((N, LBL), jnp.float32),
    )


def kernel(x, edge_index, W1, W2, b1, W3, b3):
    H = W1.shape[1]
    LBL = W3.shape[1]
    src = edge_index[0]
    dst = edge_index[1]
    pad = EP - E

    junk = jnp.full((2, pad), N + 100, jnp.int32)
    e_deg = jnp.concatenate([edge_index, junk], axis=1)
    e_deg = e_deg.reshape(2, TILES, KJ, CH)
    degp = _deg_kernel()(e_deg)
    dego = degp[0, :N, 0].reshape(N, 1)
    degi = degp[1, :N, 0].reshape(N, 1)

    y0, y1 = _scale_kernel(x, x, dego)
    y2z = jnp.concatenate(
        [y0, y1, jnp.zeros((LANES, HALF), jnp.float32)], axis=0)
    zrow = 2 * N

    padi = jnp.full((pad,), zrow, jnp.int32)
    srcg = jnp.stack([
        jnp.concatenate([src, padi]),
        jnp.concatenate([src + N, padi]),
    ]).reshape(CORES, TILES, KJ, CH)
    dstg = jnp.concatenate(
        [dst, jnp.zeros((pad,), jnp.int32)]).reshape(TILES, KJ, CH)

    xw2b = _make_xw2(H)(x, W2, b1.reshape(1, H))
    agg = _agg_kernel()(y2z, srcg, dstg)

    out = _make_dense(H, LBL)(
        xw2b, agg, agg, degi, W1, W1, W3, b3.reshape(1, LBL))
    return out


# speedup vs baseline: 1.3207x; 1.3207x over previous
import functools

import jax
import jax.numpy as jnp
from jax import lax
from jax.experimental import pallas as pl
from jax.experimental.pallas import tpu as pltpu
from jax.experimental.pallas import tpu_sc as plsc

CORES = 2
TILES = 16
LANES = 16
CH = 128

N = 10000
D = 256
E = 160000
BLK = 1000

EPT = -(-E // (TILES * CH)) * CH
KJ = EPT // CH
EP = EPT * TILES
NROWS = 10240
SLAB = NROWS // TILES
ZR = 32
HALF = D // 2


def _zero_vmem(buf, nrows, value=0.0):
    def fill(i, carry):
        for k in range(HALF // LANES):
            buf[i, pl.ds(k * LANES, LANES)] = jnp.full(
                (LANES,), value, jnp.float32)
        return carry
    lax.fori_loop(0, nrows, fill, 0)


def _deg_body(e_ref, out_ref, idx_v, ones_v, zbuf, deg_sh):
    c = lax.axis_index("c")
    s = lax.axis_index("s")
    _zero_vmem(zbuf, ZR)
    _zero_vmem(ones_v, CH, 1.0)

    for t in range(SLAB // ZR):
        pltpu.sync_copy(zbuf, deg_sh.at[pl.ds(s * SLAB + t * ZR, ZR)])
    plsc.subcore_barrier()

    pltpu.sync_copy(e_ref.at[c, s], idx_v)

    def chunk(j, carry):
        pltpu.sync_copy(ones_v, deg_sh.at[idx_v.at[j]], add=True)
        return carry
    lax.fori_loop(0, KJ, chunk, 0)

    plsc.subcore_barrier()
    pltpu.sync_copy(deg_sh.at[pl.ds(s * SLAB, SLAB)],
                    out_ref.at[c, pl.ds(s * SLAB, SLAB)])


@functools.cache
def _deg_kernel():
    mesh = plsc.VectorSubcoreMesh(
        core_axis_name="c", subcore_axis_name="s",
        num_cores=CORES, num_subcores=TILES)
    return pl.kernel(
        _deg_body,
        out_type=jax.ShapeDtypeStruct((CORES, NROWS, HALF), jnp.float32),
        mesh=mesh,
        scratch_types=[
            pltpu.VMEM((KJ, CH), jnp.int32),
            pltpu.VMEM((CH, HALF), jnp.float32),
            pltpu.VMEM((ZR, HALF), jnp.float32),
            pltpu.VMEM_SHARED((NROWS, HALF), jnp.float32),
        ],
    )


def _scale_body(xa_ref, xb_ref, dego_ref, y0_ref, y1_ref):
    s = lax.rsqrt(jnp.maximum(dego_ref[...], 1.0))
    y0_ref[...] = xa_ref[...] * s
    y1_ref[...] = xb_ref[...] * s


_scale_kernel = pl.pallas_call(
    _scale_body,
    grid=(N // BLK,),
    in_specs=[
        pl.BlockSpec((BLK, HALF), lambda i: (i, 0)),
        pl.BlockSpec((BLK, HALF), lambda i: (i, 1)),
        pl.BlockSpec((BLK, 1), lambda i: (i, 0)),
    ],
    out_specs=[
        pl.BlockSpec((BLK, HALF), lambda i: (i, 0)),
        pl.BlockSpec((BLK, HALF), lambda i: (i, 0)),
    ],
    out_shape=[
        jax.ShapeDtypeStruct((N, HALF), jnp.float32),
        jax.ShapeDtypeStruct((N, HALF), jnp.float32),
    ],
)


def _agg_body(y_ref, src_ref, dst_ref, out_ref,
              src_v, dst_v, rows_v, zbuf, agg_sh, sem):
    c = lax.axis_index("c")
    s = lax.axis_index("s")

    _zero_vmem(zbuf, ZR)

    for t in range(SLAB // ZR):
        pltpu.sync_copy(zbuf, agg_sh.at[pl.ds(s * SLAB + t * ZR, ZR)])
    plsc.subcore_barrier()

    pltpu.sync_copy(src_ref.at[c, s], src_v)
    pltpu.sync_copy(dst_ref.at[s], dst_v)

    def chunk(j, carry):
        pltpu.async_copy(y_ref.at[src_v.at[j]], rows_v, sem).wait()
        pltpu.sync_copy(rows_v, agg_sh.at[dst_v.at[j]], add=True)
        return carry
    lax.fori_loop(0, KJ, chunk, 0)

    plsc.subcore_barrier()
    pltpu.sync_copy(agg_sh.at[pl.ds(s * SLAB, SLAB)],
                    out_ref.at[c, pl.ds(s * SLAB, SLAB)])


@functools.cache
def _agg_kernel():
    mesh = plsc.VectorSubcoreMesh(
        core_axis_name="c", subcore_axis_name="s",
        num_cores=CORES, num_subcores=TILES)
    return pl.kernel(
        _agg_body,
        out_type=jax.ShapeDtypeStruct((CORES, NROWS, HALF), jnp.float32),
        mesh=mesh,
        scratch_types=[
            pltpu.VMEM((KJ, CH), jnp.int32),
            pltpu.VMEM((KJ, CH), jnp.int32),
            pltpu.VMEM((CH, HALF), jnp.float32),
            pltpu.VMEM((ZR, HALF), jnp.float32),
            pltpu.VMEM_SHARED((NROWS, HALF), jnp.float32),
            pltpu.SemaphoreType.DMA,
        ],
    )


def _dense_body(x_ref, a0_ref, a1_ref, degi_ref, w1a_ref, w1b_ref,
                w2_ref, b1_ref, w3_ref, b3_ref, out_ref):
    s = lax.rsqrt(jnp.maximum(degi_ref[...], 1.0))
    h = jnp.dot(a0_ref[0] * s, w1a_ref[...],
                preferred_element_type=jnp.float32)
    h += jnp.dot(a1_ref[0] * s, w1b_ref[...],
                 preferred_element_type=jnp.float32)
    h += jnp.dot(x_ref[...], w2_ref[...], preferred_element_type=jnp.float32)
    h += b1_ref[...]
    h = jnp.where(h >= 0.0, h, 0.2 * h)
    logits = jnp.dot(h, w3_ref[...], preferred_element_type=jnp.float32)
    logits += b3_ref[...]
    m = jnp.max(logits, axis=-1, keepdims=True)
    e = jnp.exp(logits - m)
    out_ref[...] = e / jnp.sum(e, axis=-1, keepdims=True)


def _make_dense(H, LBL):
    return pl.pallas_call(
        _dense_body,
        grid=(N // BLK,),
        in_specs=[
            pl.BlockSpec((BLK, D), lambda i: (i, 0)),
            pl.BlockSpec((1, BLK, HALF), lambda i: (0, i, 0)),
            pl.BlockSpec((1, BLK, HALF), lambda i: (1, i, 0)),
            pl.BlockSpec((BLK, 1), lambda i: (i, 0)),
            pl.BlockSpec((HALF, H), lambda i: (0, 0)),
            pl.BlockSpec((HALF, H), lambda i: (1, 0)),
            pl.BlockSpec((D, H), lambda i: (0, 0)),
            pl.BlockSpec((1, H), lambda i: (0, 0)),
            pl.BlockSpec((H, LBL), lambda i: (0, 0)),
            pl.BlockSpec((1, LBL), lambda i: (0, 0)),
        ],
        out_specs=pl.BlockSpec((BLK, LBL), lambda i: (i, 0)),
        out_shape=jax.ShapeDtypeStruct((N, LBL), jnp.float32),
    )


def kernel(x, edge_index, W1, W2, b1, W3, b3):
    H = W1.shape[1]
    LBL = W3.shape[1]
    src = edge_index[0]
    dst = edge_index[1]
    pad = EP - E

    junk = jnp.full((2, pad), N + 100, jnp.int32)
    e_deg = jnp.concatenate([edge_index, junk], axis=1)
    e_deg = e_deg.reshape(2, TILES, KJ, CH)
    degp = _deg_kernel()(e_deg)
    dego = degp[0, :N, 0].reshape(N, 1)
    degi = degp[1, :N, 0].reshape(N, 1)

    y0, y1 = _scale_kernel(x, x, dego)
    y2z = jnp.concatenate(
        [y0, y1, jnp.zeros((LANES, HALF), jnp.float32)], axis=0)
    zrow = 2 * N

    padi = jnp.full((pad,), zrow, jnp.int32)
    srcg = jnp.stack([
        jnp.concatenate([src, padi]),
        jnp.concatenate([src + N, padi]),
    ]).reshape(CORES, TILES, KJ, CH)
    dstg = jnp.concatenate(
        [dst, jnp.zeros((pad,), jnp.int32)]).reshape(TILES, KJ, CH)

    agg = _agg_kernel()(y2z, srcg, dstg)

    out = _make_dense(H, LBL)(
        x, agg, agg, degi, W1, W1, W2,
        b1.reshape(1, H), W3, b3.reshape(1, LBL))
    return out
